# baseline (speedup 1.0000x reference)
import jax
import jax.numpy as jnp
from jax import lax
from jax.experimental import pallas as pl
from jax.experimental.pallas import tpu as pltpu
from jax.experimental.pallas import tpu_sc as plsc

B = 16384
F = 128
P = 26
NM = 25
OUT_W = F + 2

NC = 2
NS = 16
NW = NC * NS
COLS = B // NW
GRP = 11


def _sc_body(ft_hbm, pt_hbm, out_hbm, ft_v, pt_v, pair_v, sem_in, sem_out):
    wid = lax.axis_index("s") * NC + lax.axis_index("c")
    base = wid * COLS

    in_f = pltpu.async_copy(ft_hbm.at[:, :, pl.ds(base, COLS)], ft_v, sem_in)
    in_p = pltpu.async_copy(pt_hbm.at[:, :, pl.ds(base, COLS)], pt_v, sem_in)
    in_p.wait()
    in_f.wait()

    copies = [
        pltpu.async_copy(
            ft_v,
            out_hbm.at[pl.ds(i, 1), pl.ds(0, F), pl.ds(base, COLS)],
            sem_out,
        )
        for i in range(NM)
    ]

    pending = {}
    for i in range(NM):
        q = i % GRP
        if i >= GRP:
            pending.pop(i - GRP).wait()
        for d in range(2):
            for k in range(COLS // 16):
                pair_v[q, d, pl.ds(16 * k, 16)] = pt_v[0, i + d, pl.ds(16 * k, 16)]
        pending[i] = pltpu.async_copy(
            pair_v.at[pl.ds(q, 1), :, :],
            out_hbm.at[pl.ds(i, 1), pl.ds(F, 2), pl.ds(base, COLS)],
            sem_out,
        )
    for d in pending.values():
        d.wait()
    for d in copies:
        d.wait()


@jax.jit
def kernel(features, parameters):
    ft = features.T.reshape(1, F, B)
    pt = parameters.T.reshape(1, P, B)
    mesh = plsc.VectorSubcoreMesh(
        core_axis_name="c", subcore_axis_name="s", num_cores=NC, num_subcores=NS
    )
    run = pl.kernel(
        _sc_body,
        out_type=jax.ShapeDtypeStruct((NM, OUT_W, B), jnp.float32),
        mesh=mesh,
        scratch_types=[
            pltpu.VMEM((1, F, COLS), jnp.float32),
            pltpu.VMEM((1, P, COLS), jnp.float32),
            pltpu.VMEM((GRP, 2, COLS), jnp.float32),
            pltpu.SemaphoreType.DMA,
            pltpu.SemaphoreType.DMA,
        ],
    )
    out = run(ft, pt)
    return out.transpose(2, 0, 1)


# speedup vs baseline: 1.0269x; 1.0269x over previous
import jax
import jax.numpy as jnp
from jax import lax
from jax.experimental import pallas as pl
from jax.experimental.pallas import tpu as pltpu
from jax.experimental.pallas import tpu_sc as plsc

B = 16384
F = 128
P = 26
NM = 25
OUT_W = F + 2

NC = 2
NS = 16
NW = NC * NS
COLS = B // NW


def _sc_body(ft_hbm, out_hbm, ft_v, sem_in, sem_out):
    wid = lax.axis_index("s") * NC + lax.axis_index("c")
    base = wid * COLS

    pltpu.async_copy(ft_hbm.at[:, :, pl.ds(base, COLS)], ft_v, sem_in).wait()

    copies = [
        pltpu.async_copy(
            ft_v,
            out_hbm.at[pl.ds(i, 1), pl.ds(0, F), pl.ds(base, COLS)],
            sem_out,
        )
        for i in range(NM)
    ]
    for d in copies:
        d.wait()


def _tc_pairs_body(_, pt_ref, out_ref, pair_v, sem):
    for i in range(NM):
        pair_v[i, 0, :] = pt_ref[0, i, :]
        pair_v[i, 1, :] = pt_ref[0, i + 1, :]
    pltpu.make_async_copy(pair_v, out_ref.at[:, pl.ds(F, 2), :], sem).start()
    pltpu.make_async_copy(pair_v, out_ref.at[:, pl.ds(F, 2), :], sem).wait()


@jax.jit
def kernel(features, parameters):
    ft = features.T.reshape(1, F, B)
    pt = parameters.T.reshape(1, P, B)
    mesh = plsc.VectorSubcoreMesh(
        core_axis_name="c", subcore_axis_name="s", num_cores=NC, num_subcores=NS
    )
    sc_run = pl.kernel(
        _sc_body,
        out_type=jax.ShapeDtypeStruct((NM, OUT_W, B), jnp.float32),
        mesh=mesh,
        scratch_types=[
            pltpu.VMEM((1, F, COLS), jnp.float32),
            pltpu.SemaphoreType.DMA,
            pltpu.SemaphoreType.DMA,
        ],
    )
    feat_out = sc_run(ft)

    out = pl.pallas_call(
        _tc_pairs_body,
        in_specs=[
            pl.BlockSpec(memory_space=pl.ANY),
            pl.BlockSpec((1, P, B), lambda: (0, 0, 0)),
        ],
        out_specs=pl.BlockSpec(memory_space=pl.ANY),
        out_shape=jax.ShapeDtypeStruct((NM, OUT_W, B), jnp.float32),
        scratch_shapes=[
            pltpu.VMEM((NM, 2, B), jnp.float32),
            pltpu.SemaphoreType.DMA,
        ],
        input_output_aliases={0: 0},
    )(feat_out, pt)
    return out.transpose(2, 0, 1)
